# baseline (speedup 1.0000x reference)
import functools

import jax
import jax.numpy as jnp
from jax import lax
from jax.experimental import pallas as pl
from jax.experimental.pallas import tpu as pltpu
from jax.experimental.pallas import tpu_sc as plsc

N = 10000
K = 16
NEG_SLOPE = 0.01
EPS = 1e-5

QB = 256
KEY_CH = 1024


def _knn_kernel_body(ch, nvalid, nch, ka_ref, qat_ref, out_ref, d_ref):
    b = pl.program_id(0)
    inf = jnp.float32(jnp.inf)
    big = jnp.int32(1 << 30)

    nkey = ch * nch
    dot = lax.dot_general(
        ka_ref[...], qat_ref[0, :, :], (((1,), (0,)), ((), ())),
        preferred_element_type=jnp.float32)
    qsq = qat_ref[0, 4:5, :]
    ksq = ka_ref[:, 3:4]
    d = (qsq - 2.0 * dot) + ksq
    kio_full = lax.broadcasted_iota(jnp.int32, (nkey, QB), 0)
    qid_full = b * QB + lax.broadcasted_iota(jnp.int32, (nkey, QB), 1)
    d_ref[...] = jnp.where((kio_full == qid_full) | (kio_full >= nvalid), inf, d)

    def extract(t, prev_idx):
        def sweep_chunk(c, carry):
            w, idx = carry
            s = c * ch
            kio = s + lax.broadcasted_iota(jnp.int32, (ch, QB), 0)
            dc = d_ref[pl.ds(s, ch), :]
            dc = jnp.where(kio == prev_idx, inf, dc)
            d_ref[pl.ds(s, ch), :] = dc
            cmin = jnp.min(dc, axis=0, keepdims=True)
            cidx = jnp.min(jnp.where(dc == cmin, kio, big), axis=0, keepdims=True)
            upd = (cmin < w) | ((cmin == w) & (cidx < idx))
            return jnp.where(upd, cmin, w), jnp.where(upd, cidx, idx)

        _, idx = lax.fori_loop(
            0, nch, sweep_chunk,
            (jnp.full((1, QB), inf, jnp.float32), jnp.full((1, QB), big, jnp.int32)),
        )
        out_ref[0, pl.ds(t, 1), :] = idx
        return idx

    lax.fori_loop(0, K, extract, jnp.full((1, QB), -1, jnp.int32))


def _knn_topk(ka, qat, nvalid, *, interpret=False):
    nkey = ka.shape[0]
    nb = qat.shape[0]
    ch = min(KEY_CH, nkey)
    assert nkey % ch == 0
    nch = nkey // ch
    body = functools.partial(_knn_kernel_body, ch, nvalid, nch)
    return pl.pallas_call(
        body,
        grid=(nb,),
        in_specs=[
            pl.BlockSpec((nkey, 8), lambda b: (0, 0)),
            pl.BlockSpec((1, 8, QB), lambda b: (b, 0, 0)),
        ],
        out_specs=pl.BlockSpec((1, K, QB), lambda b: (b, 0, 0)),
        out_shape=jax.ShapeDtypeStruct((nb, K, QB), jnp.int32),
        scratch_shapes=[pltpu.VMEM((nkey, QB), jnp.float32)],
        compiler_params=pltpu.CompilerParams(
            dimension_semantics=("parallel",)),
        interpret=interpret,
    )(ka, qat)


def _uv_kernel_body(x_ref, w_ref, b_ref, uv_ref):
    res = jnp.dot(x_ref[...], w_ref[...], preferred_element_type=jnp.float32)
    uv_ref[...] = res + b_ref[0:1, :]


def _uv(x8, wcat, brow, *, interpret=False):
    m = x8.shape[0]
    return pl.pallas_call(
        _uv_kernel_body,
        out_shape=jax.ShapeDtypeStruct((m, 128), jnp.float32),
        interpret=interpret,
    )(x8, wcat, brow)


def _gather_rows(v, idx):
    nidx = idx.shape[0]
    nw = 32
    b_per_w = nidx // nw
    chunk = 1000
    nloop = b_per_w // chunk
    mesh = plsc.VectorSubcoreMesh(core_axis_name="c", subcore_axis_name="s")

    @functools.partial(
        pl.kernel,
        mesh=mesh,
        out_type=jax.ShapeDtypeStruct((nidx, 128), jnp.float32),
        scratch_types=[
            pltpu.VMEM((chunk,), jnp.int32),
            pltpu.VMEM((chunk, 128), jnp.float32),
            pltpu.SemaphoreType.DMA,
        ],
    )
    def gk(v_hbm, idx_hbm, out_hbm, idx_v, rows_v, sem):
        wid = lax.axis_index("s") * 2 + lax.axis_index("c")

        @pl.loop(0, nloop)
        def _(c):
            base = wid * b_per_w + c * chunk
            pltpu.sync_copy(idx_hbm.at[pl.ds(base, chunk)], idx_v)
            pltpu.async_copy(v_hbm.at[idx_v], rows_v, sem).wait()
            pltpu.sync_copy(rows_v, out_hbm.at[pl.ds(base, chunk)])

    return gk(v, idx)


def _stats_kernel_body(g_ref, u_ref, s_ref):
    i = pl.program_id(0)

    @pl.when(i == 0)
    def _():
        s_ref[...] = jnp.zeros_like(s_ref)

    m = g_ref[...][:, :, 64:] + u_ref[...][:, None, :64]
    s0 = jnp.sum(m, axis=(0, 1))
    s1 = jnp.sum(m * m, axis=(0, 1))
    s_ref[0:1, :] += s0[None, :]
    s_ref[1:2, :] += s1[None, :]


def _stats(g3, u, *, interpret=False):
    n = g3.shape[0]
    nb = 400
    return pl.pallas_call(
        _stats_kernel_body,
        grid=(n // nb,),
        in_specs=[
            pl.BlockSpec((nb, K, 128), lambda i: (i, 0, 0)),
            pl.BlockSpec((nb, 128), lambda i: (i, 0)),
        ],
        out_specs=pl.BlockSpec((8, 64), lambda i: (0, 0)),
        out_shape=jax.ShapeDtypeStruct((8, 64), jnp.float32),
        interpret=interpret,
    )(g3, u)


def _apply_kernel_body(g_ref, u_ref, p_ref, o_ref):
    scale = p_ref[0, :]
    shift = p_ref[1, :]
    m = g_ref[...][:, :, 64:] + u_ref[...][:, None, :64]
    m = m * scale + shift
    m = jnp.where(m > 0, m, NEG_SLOPE * m)
    o_ref[...] = jnp.max(m, axis=1)


def _apply(g3, u, params, *, interpret=False):
    n = g3.shape[0]
    nb = 400
    return pl.pallas_call(
        _apply_kernel_body,
        grid=(n // nb,),
        in_specs=[
            pl.BlockSpec((nb, K, 128), lambda i: (i, 0, 0)),
            pl.BlockSpec((nb, 128), lambda i: (i, 0)),
            pl.BlockSpec((8, 64), lambda i: (0, 0)),
        ],
        out_specs=pl.BlockSpec((nb, 64), lambda i: (i, 0)),
        out_shape=jax.ShapeDtypeStruct((n, 64), jnp.float32),
        interpret=interpret,
    )(g3, u, params)


def kernel(x, W1, b1, g1, be1):
    n = x.shape[0]
    nkey = 10240
    nb = 40

    sq = jnp.sum(x * x, axis=1)
    ka = jnp.zeros((nkey, 8), jnp.float32)
    ka = ka.at[:n, :3].set(x).at[:n, 3].set(sq)

    qa = jnp.zeros((nb * QB, 8), jnp.float32)
    qa = qa.at[:n, :3].set(x).at[:n, 4].set(sq)
    qat = qa.reshape(nb, QB, 8).transpose(0, 2, 1)

    nbr = _knn_topk(ka, qat, n)
    idx = nbr.transpose(0, 2, 1).reshape(nb * QB, K)[:n]
    idx_flat = idx.reshape(-1).astype(jnp.int32)

    wa = W1[:, :3]
    wb = W1[:, 3:]
    mpad = 10016
    x8 = jnp.zeros((mpad, 8), jnp.float32).at[:n, :3].set(x)
    wcat = jnp.zeros((8, 128), jnp.float32)
    wcat = wcat.at[:3, :64].set((wa - wb).T).at[:3, 64:].set(wb.T)
    brow = jnp.zeros((8, 128), jnp.float32).at[0, :64].set(b1)
    uv = _uv(x8, wcat, brow)
    u = uv[:n]

    g = _gather_rows(uv, idx_flat)
    g3 = g.reshape(n, K, 128)

    sums = _stats(g3, u)
    cnt = jnp.float32(n * K)
    mu = sums[0] / cnt
    var = jnp.maximum(sums[1] / cnt - mu * mu, 0.0)
    scale = g1 / jnp.sqrt(var + EPS)
    shift = be1 - mu * scale
    params = jnp.zeros((8, 64), jnp.float32).at[0].set(scale).at[1].set(shift)

    return _apply(g3, u, params)


# speedup vs baseline: 1.8093x; 1.8093x over previous
import functools

import jax
import jax.numpy as jnp
import numpy as np
from jax import lax
from jax.experimental import pallas as pl
from jax.experimental.shard_map import shard_map
from jax.sharding import Mesh, PartitionSpec as P
from jax.experimental.pallas import tpu as pltpu
from jax.experimental.pallas import tpu_sc as plsc

N = 10000
K = 16
NEG_SLOPE = 0.01
EPS = 1e-5

QB = 256
KEY_CH = 1024


def _knn_kernel_body(ch, nvalid, nch, ka_ref, qat_ref, out_ref, d_ref):
    inf = jnp.float32(jnp.inf)
    big = jnp.int32(1 << 30)

    nkey = ch * nch
    dot = lax.dot_general(
        ka_ref[...], qat_ref[0, :, :], (((1,), (0,)), ((), ())),
        preferred_element_type=jnp.float32)
    qsq = qat_ref[0, 4:5, :]
    ksq = ka_ref[:, 3:4]
    d = (qsq - 2.0 * dot) + ksq
    qid = qat_ref[0, 5:6, :].astype(jnp.int32)
    kio_full = lax.broadcasted_iota(jnp.int32, (nkey, QB), 0)
    d_ref[...] = jnp.where((kio_full == qid) | (kio_full >= nvalid), inf, d)

    def extract(t, prev_idx):
        def sweep_chunk(c, carry):
            w, idx = carry
            s = c * ch
            kio = s + lax.broadcasted_iota(jnp.int32, (ch, QB), 0)
            dc = d_ref[pl.ds(s, ch), :]
            dc = jnp.where(kio == prev_idx, inf, dc)
            d_ref[pl.ds(s, ch), :] = dc
            cmin = jnp.min(dc, axis=0, keepdims=True)
            cidx = jnp.min(jnp.where(dc == cmin, kio, big), axis=0, keepdims=True)
            upd = (cmin < w) | ((cmin == w) & (cidx < idx))
            return jnp.where(upd, cmin, w), jnp.where(upd, cidx, idx)

        _, idx = lax.fori_loop(
            0, nch, sweep_chunk,
            (jnp.full((1, QB), inf, jnp.float32), jnp.full((1, QB), big, jnp.int32)),
        )
        out_ref[0, pl.ds(t, 1), :] = idx
        return idx

    lax.fori_loop(0, K, extract, jnp.full((1, QB), -1, jnp.int32))


def _knn_topk(ka, qat, nvalid, *, interpret=False):
    nkey = ka.shape[0]
    nb = qat.shape[0]
    ch = min(KEY_CH, nkey)
    assert nkey % ch == 0
    nch = nkey // ch
    body = functools.partial(_knn_kernel_body, ch, nvalid, nch)
    return pl.pallas_call(
        body,
        grid=(nb,),
        in_specs=[
            pl.BlockSpec((nkey, 8), lambda b: (0, 0)),
            pl.BlockSpec((1, 8, QB), lambda b: (b, 0, 0)),
        ],
        out_specs=pl.BlockSpec((1, K, QB), lambda b: (b, 0, 0)),
        out_shape=jax.ShapeDtypeStruct((nb, K, QB), jnp.int32),
        scratch_shapes=[pltpu.VMEM((nkey, QB), jnp.float32)],
        compiler_params=pltpu.CompilerParams(
            dimension_semantics=("parallel",)),
        interpret=interpret,
    )(ka, qat)


def _uv_kernel_body(x_ref, w_ref, b_ref, uv_ref):
    res = jnp.dot(x_ref[...], w_ref[...], preferred_element_type=jnp.float32)
    uv_ref[...] = res + b_ref[0:1, :]


def _uv(x8, wcat, brow, *, interpret=False):
    m = x8.shape[0]
    return pl.pallas_call(
        _uv_kernel_body,
        out_shape=jax.ShapeDtypeStruct((m, 128), jnp.float32),
        interpret=interpret,
    )(x8, wcat, brow)


def _gather_rows(v, idx):
    nidx = idx.shape[0]
    nw = 32
    chunk = 1000
    nchunk = nidx // chunk
    nloop = -(-nchunk // nw)
    mesh = plsc.VectorSubcoreMesh(core_axis_name="c", subcore_axis_name="s")

    @functools.partial(
        pl.kernel,
        mesh=mesh,
        out_type=jax.ShapeDtypeStruct((nidx, 128), jnp.float32),
        scratch_types=[
            pltpu.VMEM((chunk,), jnp.int32),
            pltpu.VMEM((chunk, 128), jnp.float32),
            pltpu.SemaphoreType.DMA,
        ],
    )
    def gk(v_hbm, idx_hbm, out_hbm, idx_v, rows_v, sem):
        wid = lax.axis_index("s") * 2 + lax.axis_index("c")

        @pl.loop(0, nloop)
        def _(c):
            j = c * nw + wid

            @pl.when(j < nchunk)
            def _():
                base = j * chunk
                pltpu.sync_copy(idx_hbm.at[pl.ds(base, chunk)], idx_v)
                pltpu.async_copy(v_hbm.at[idx_v], rows_v, sem).wait()
                pltpu.sync_copy(rows_v, out_hbm.at[pl.ds(base, chunk)])

    return gk(v, idx)


def _stats_kernel_body(g_ref, u_ref, s_ref):
    i = pl.program_id(0)

    @pl.when(i == 0)
    def _():
        s_ref[...] = jnp.zeros_like(s_ref)

    m = g_ref[...][:, :, 64:] + u_ref[...][:, None, :64]
    s0 = jnp.sum(m, axis=(0, 1))
    s1 = jnp.sum(m * m, axis=(0, 1))
    s_ref[0:1, :] += s0[None, :]
    s_ref[1:2, :] += s1[None, :]


def _stats(g3, u, *, interpret=False):
    n = g3.shape[0]
    nb = 200
    return pl.pallas_call(
        _stats_kernel_body,
        grid=(n // nb,),
        in_specs=[
            pl.BlockSpec((nb, K, 128), lambda i: (i, 0, 0)),
            pl.BlockSpec((nb, 128), lambda i: (i, 0)),
        ],
        out_specs=pl.BlockSpec((8, 64), lambda i: (0, 0)),
        out_shape=jax.ShapeDtypeStruct((8, 64), jnp.float32),
        interpret=interpret,
    )(g3, u)


def _apply_kernel_body(g_ref, u_ref, p_ref, o_ref):
    scale = p_ref[0, :]
    shift = p_ref[1, :]
    m = g_ref[...][:, :, 64:] + u_ref[...][:, None, :64]
    m = m * scale + shift
    m = jnp.where(m > 0, m, NEG_SLOPE * m)
    o_ref[...] = jnp.max(m, axis=1)


def _apply(g3, u, params, *, interpret=False):
    n = g3.shape[0]
    nb = 200
    return pl.pallas_call(
        _apply_kernel_body,
        grid=(n // nb,),
        in_specs=[
            pl.BlockSpec((nb, K, 128), lambda i: (i, 0, 0)),
            pl.BlockSpec((nb, 128), lambda i: (i, 0)),
            pl.BlockSpec((8, 64), lambda i: (0, 0)),
        ],
        out_specs=pl.BlockSpec((nb, 64), lambda i: (i, 0)),
        out_shape=jax.ShapeDtypeStruct((n, 64), jnp.float32),
        interpret=interpret,
    )(g3, u, params)


def kernel(x, W1, b1, g1, be1):
    n = x.shape[0]
    nkey = 10240

    devs = jax.devices()
    ndev = 2 if len(devs) >= 2 and n % 2 == 0 else 1
    npd = n // ndev
    slots_pd = -(-npd // QB) * QB
    nb_pd = slots_pd // QB
    tot_slots = ndev * slots_pd

    sq = jnp.sum(x * x, axis=1)
    ka = jnp.zeros((nkey, 8), jnp.float32)
    ka = ka.at[:n, :3].set(x).at[:n, 3].set(sq)

    s = jnp.arange(tot_slots)
    node = (s // slots_pd) * npd + (s % slots_pd)
    nodec = jnp.minimum(node, n - 1)
    qa = jnp.zeros((tot_slots, 8), jnp.float32)
    qa = qa.at[:, :3].set(x[nodec]).at[:, 4].set(sq[nodec])
    qa = qa.at[:, 5].set(node.astype(jnp.float32))
    qat = qa.reshape(ndev * nb_pd, QB, 8).transpose(0, 2, 1)

    wa = W1[:, :3]
    wb = W1[:, 3:]
    mpad = 10016
    x8 = jnp.zeros((mpad, 8), jnp.float32).at[:n, :3].set(x)
    wcat = jnp.zeros((8, 128), jnp.float32)
    wcat = wcat.at[:3, :64].set((wa - wb).T).at[:3, 64:].set(wb.T)
    brow = jnp.zeros((8, 128), jnp.float32).at[0, :64].set(b1)

    cnt = jnp.float32(n * K)

    def shard_fn(ka, qat_l, x8, wcat, brow, g1, be1):
        dix = lax.axis_index("d")
        nbr_l = _knn_topk(ka, qat_l, n)
        idx_l = (nbr_l.transpose(0, 2, 1).reshape(slots_pd, K)[:npd]
                 .reshape(-1).astype(jnp.int32))
        uv = _uv(x8, wcat, brow)
        u_l = lax.dynamic_slice(uv, (dix * npd, 0), (npd, 128))
        g3 = _gather_rows(uv, idx_l).reshape(npd, K, 128)
        sums = lax.psum(_stats(g3, u_l), "d")
        mu = sums[0] / cnt
        var = jnp.maximum(sums[1] / cnt - mu * mu, 0.0)
        scale = g1 / jnp.sqrt(var + EPS)
        shift = be1 - mu * scale
        params = jnp.zeros((8, 64), jnp.float32).at[0].set(scale).at[1].set(shift)
        return _apply(g3, u_l, params)

    mesh = Mesh(np.asarray(devs[:ndev]), ("d",))
    return shard_map(
        shard_fn, mesh=mesh,
        in_specs=(P(), P("d"), P(), P(), P(), P(), P()),
        out_specs=P("d"),
        check_rep=False,
    )(ka, qat, x8, wcat, brow, g1, be1)
